# baseline (speedup 1.0000x reference)
import functools

import jax
import jax.numpy as jnp
from jax import lax
from jax.experimental import pallas as pl
from jax.experimental.pallas import tpu as pltpu
from jax.experimental.pallas import tpu_sc as plsc

N = 128
E = 2048
B = 256
D_IN = 784
D_H = 128
LANES = 16


def _edge_scan_sc(edge_index, w):
    mesh = plsc.VectorSubcoreMesh(
        core_axis_name="c", subcore_axis_name="s", num_cores=1
    )

    @functools.partial(
        pl.kernel,
        mesh=mesh,
        out_type=jax.ShapeDtypeStruct((N,), jnp.float32),
        scratch_types=[
            pltpu.VMEM((E,), jnp.int32),
            pltpu.VMEM((E,), jnp.int32),
            pltpu.VMEM((E,), jnp.float32),
            pltpu.VMEM((N,), jnp.float32),
        ],
        compiler_params=pltpu.CompilerParams(needs_layout_passes=False),
        cost_estimate=pl.CostEstimate(
            flops=2_000_000_000, bytes_accessed=64 << 20, transcendentals=0
        ),
    )
    def scan_kernel(edge_hbm, w_hbm, c_hbm, src_v, tgt_v, w_v, c_v):
        cid = lax.axis_index("c")
        sid = lax.axis_index("s")

        @pl.when(jnp.logical_and(cid == 0, sid == 0))
        def _():
            pltpu.sync_copy(edge_hbm.at[0], src_v)
            pltpu.sync_copy(edge_hbm.at[1], tgt_v)
            pltpu.sync_copy(w_hbm, w_v)
            init = jnp.full((LANES,), 1.0 / N, jnp.float32)
            for j in range(N // LANES):
                c_v[pl.ds(j * LANES, LANES)] = init
            lane0 = lax.iota(jnp.int32, LANES) == 0
            dnums = lax.GatherDimensionNumbers(
                offset_dims=(), collapsed_slice_dims=(0,), start_index_map=(0,)
            )

            def bcast(vec, j):
                idx = jnp.full((LANES, 1), j, jnp.int32)
                return lax.gather(
                    vec, idx, dnums, (1,),
                    mode=lax.GatherScatterMode.PROMISE_IN_BOUNDS,
                )

            def body(i, carry):
                base = (E // LANES - 1 - i) * LANES
                s16 = src_v[pl.ds(base, LANES)]
                t16 = tgt_v[pl.ds(base, LANES)]
                w16 = w_v[pl.ds(base, LANES)]
                for j in range(LANES - 1, -1, -1):
                    t_b = bcast(t16, j)
                    s_b = bcast(s16, j)
                    w_b = bcast(w16, j)
                    ct = plsc.load_gather(c_v, [t_b])
                    plsc.addupdate_scatter(c_v, [s_b], w_b * ct, mask=lane0)
                return carry

            lax.fori_loop(0, E // LANES, body, 0)
            pltpu.sync_copy(c_v, c_hbm)

    return scan_kernel(edge_index, w)


NB = 64


def _tanh_terms_tc(x, W, b):
    def body(x_ref, w_ref, b_ref, t_ref):
        for k in range(NB):
            t_ref[k] = jnp.tanh(
                jnp.dot(x_ref[...], w_ref[k], preferred_element_type=jnp.float32)
                + b_ref[k][None, :]
            ).astype(jnp.bfloat16)

    return pl.pallas_call(
        body,
        grid=(N // NB,),
        in_specs=[
            pl.BlockSpec((B, D_IN), lambda i: (0, 0)),
            pl.BlockSpec((NB, D_IN, D_H), lambda i: (i, 0, 0)),
            pl.BlockSpec((NB, D_H), lambda i: (i, 0)),
        ],
        out_specs=pl.BlockSpec((NB, B, D_H), lambda i: (i, 0, 0)),
        out_shape=jax.ShapeDtypeStruct((N, B, D_H), jnp.bfloat16),
        compiler_params=pltpu.CompilerParams(
            dimension_semantics=("arbitrary",),
        ),
    )(x, W, b)


def _weighted_reduce_tc(c, T):
    def body(c_ref, t_ref, out_ref):
        i = pl.program_id(0)

        @pl.when(i == 0)
        def _():
            out_ref[...] = jnp.zeros_like(out_ref)

        acc = out_ref[...]
        for k in range(NB):
            acc = acc + c_ref[i * NB + k] * t_ref[k].astype(jnp.float32)
        out_ref[...] = acc

    return pl.pallas_call(
        body,
        grid=(N // NB,),
        in_specs=[
            pl.BlockSpec(memory_space=pltpu.SMEM),
            pl.BlockSpec((NB, B, D_H), lambda i: (i, 0, 0)),
        ],
        out_specs=pl.BlockSpec((B, D_H), lambda i: (0, 0)),
        out_shape=jax.ShapeDtypeStruct((B, D_H), jnp.float32),
        compiler_params=pltpu.CompilerParams(
            dimension_semantics=("arbitrary",),
        ),
    )(c, T)


def kernel(x, W, b, edge_index, edge_weights):
    T = _tanh_terms_tc(x, W, b)
    c = _edge_scan_sc(edge_index, edge_weights)
    return _weighted_reduce_tc(c, T)


# speedup vs baseline: 1.1067x; 1.1067x over previous
import functools

import jax
import jax.numpy as jnp
from jax import lax
from jax.experimental import pallas as pl
from jax.experimental.pallas import tpu as pltpu
from jax.experimental.pallas import tpu_sc as plsc

N = 128
E = 2048
B = 256
D_IN = 784
D_H = 128
LANES = 16


def _edge_scan_sc(edge_index, w):
    mesh = plsc.VectorSubcoreMesh(
        core_axis_name="c", subcore_axis_name="s", num_cores=1
    )

    @functools.partial(
        pl.kernel,
        mesh=mesh,
        out_type=jax.ShapeDtypeStruct((N,), jnp.float32),
        scratch_types=[
            pltpu.VMEM((E,), jnp.int32),
            pltpu.VMEM((E,), jnp.int32),
            pltpu.VMEM((E,), jnp.float32),
            pltpu.VMEM((N,), jnp.float32),
            pltpu.SemaphoreType.DMA,
        ],
        compiler_params=pltpu.CompilerParams(needs_layout_passes=False),
    )
    def scan_kernel(edge_hbm, w_hbm, c_hbm, src_v, tgt_v, w_v, c_v, sem):
        cid = lax.axis_index("c")
        sid = lax.axis_index("s")

        @pl.when(jnp.logical_and(cid == 0, sid == 0))
        def _():
            cp1 = pltpu.async_copy(edge_hbm.at[0], src_v, sem)
            cp2 = pltpu.async_copy(edge_hbm.at[1], tgt_v, sem)
            cp3 = pltpu.async_copy(w_hbm, w_v, sem)
            cp1.wait()
            cp2.wait()
            cp3.wait()
            init = jnp.full((LANES,), 1.0 / N, jnp.float32)
            for j in range(N // LANES):
                c_v[pl.ds(j * LANES, LANES)] = init
            lane0 = lax.iota(jnp.int32, LANES) == 0
            dnums = lax.GatherDimensionNumbers(
                offset_dims=(), collapsed_slice_dims=(0,), start_index_map=(0,)
            )

            def bcast(vec, j):
                idx = jnp.full((LANES, 1), j, jnp.int32)
                return lax.gather(
                    vec, idx, dnums, (1,),
                    mode=lax.GatherScatterMode.PROMISE_IN_BOUNDS,
                )

            def body(i, carry):
                base = (E // LANES - 1 - i) * LANES
                s16 = src_v[pl.ds(base, LANES)]
                t16 = tgt_v[pl.ds(base, LANES)]
                w16 = w_v[pl.ds(base, LANES)]
                for j in range(LANES - 1, -1, -1):
                    t_b = bcast(t16, j)
                    s_b = bcast(s16, j)
                    w_b = bcast(w16, j)
                    ct = plsc.load_gather(c_v, [t_b])
                    plsc.addupdate_scatter(c_v, [s_b], w_b * ct, mask=lane0)
                return carry

            lax.fori_loop(0, E // LANES, body, 0)
            pltpu.sync_copy(c_v, c_hbm)

    return scan_kernel(edge_index, w)


NB = 32


def _tanh_terms_tc(x, W, b):
    def body(x_ref, w_ref, b_ref, t_ref):
        for k in range(NB):
            t_ref[k] = jnp.tanh(
                jnp.dot(x_ref[...], w_ref[k], preferred_element_type=jnp.float32)
                + b_ref[k][None, :]
            ).astype(jnp.bfloat16)

    return pl.pallas_call(
        body,
        grid=(N // NB,),
        in_specs=[
            pl.BlockSpec((B, D_IN), lambda i: (0, 0)),
            pl.BlockSpec((NB, D_IN, D_H), lambda i: (i, 0, 0)),
            pl.BlockSpec((NB, D_H), lambda i: (i, 0)),
        ],
        out_specs=pl.BlockSpec((NB, B, D_H), lambda i: (i, 0, 0)),
        out_shape=jax.ShapeDtypeStruct((N, B, D_H), jnp.bfloat16),
        compiler_params=pltpu.CompilerParams(
            dimension_semantics=("arbitrary",),
        ),
    )(x, W, b)


def _weighted_reduce_tc(c, T):
    def body(c_ref, t_ref, out_ref):
        i = pl.program_id(0)

        @pl.when(i == 0)
        def _():
            out_ref[...] = jnp.zeros_like(out_ref)

        acc = out_ref[...]
        for k in range(NB):
            acc = acc + c_ref[i * NB + k] * t_ref[k].astype(jnp.float32)
        out_ref[...] = acc

    return pl.pallas_call(
        body,
        grid=(N // NB,),
        in_specs=[
            pl.BlockSpec(memory_space=pltpu.SMEM),
            pl.BlockSpec((NB, B, D_H), lambda i: (i, 0, 0)),
        ],
        out_specs=pl.BlockSpec((B, D_H), lambda i: (0, 0)),
        out_shape=jax.ShapeDtypeStruct((B, D_H), jnp.float32),
        compiler_params=pltpu.CompilerParams(
            dimension_semantics=("arbitrary",),
        ),
    )(c, T)


def kernel(x, W, b, edge_index, edge_weights):
    T = _tanh_terms_tc(x, W, b)
    c = _edge_scan_sc(edge_index, edge_weights)
    return _weighted_reduce_tc(c, T)


# speedup vs baseline: 1.1103x; 1.0033x over previous
import functools

import jax
import jax.numpy as jnp
from jax import lax
from jax.experimental import pallas as pl
from jax.experimental.pallas import tpu as pltpu
from jax.experimental.pallas import tpu_sc as plsc

N = 128
E = 2048
B = 256
D_IN = 784
D_H = 128
LANES = 16


def _edge_scan_sc(edge_index, w):
    mesh = plsc.VectorSubcoreMesh(
        core_axis_name="c", subcore_axis_name="s", num_cores=1
    )

    @functools.partial(
        pl.kernel,
        mesh=mesh,
        out_type=jax.ShapeDtypeStruct((N,), jnp.float32),
        scratch_types=[
            pltpu.VMEM((E,), jnp.int32),
            pltpu.VMEM((E,), jnp.int32),
            pltpu.VMEM((E,), jnp.float32),
            pltpu.VMEM((N,), jnp.float32),
            pltpu.SemaphoreType.DMA,
        ],
        compiler_params=pltpu.CompilerParams(needs_layout_passes=False),
    )
    def scan_kernel(edge_hbm, w_hbm, c_hbm, src_v, tgt_v, w_v, c_v, sem):
        cid = lax.axis_index("c")
        sid = lax.axis_index("s")

        @pl.when(jnp.logical_and(cid == 0, sid == 0))
        def _():
            cp1 = pltpu.async_copy(edge_hbm.at[0], src_v, sem)
            cp2 = pltpu.async_copy(edge_hbm.at[1], tgt_v, sem)
            cp3 = pltpu.async_copy(w_hbm, w_v, sem)
            cp1.wait()
            cp2.wait()
            cp3.wait()
            init = jnp.full((LANES,), 1.0 / N, jnp.float32)
            for j in range(N // LANES):
                c_v[pl.ds(j * LANES, LANES)] = init
            lane0 = lax.iota(jnp.int32, LANES) == 0
            dnums = lax.GatherDimensionNumbers(
                offset_dims=(), collapsed_slice_dims=(0,), start_index_map=(0,)
            )

            def bcast(vec, j):
                idx = jnp.full((LANES, 1), j, jnp.int32)
                return lax.gather(
                    vec, idx, dnums, (1,),
                    mode=lax.GatherScatterMode.PROMISE_IN_BOUNDS,
                )

            def body(i, carry):
                base = (E // LANES - 1 - i) * LANES
                s16 = src_v[pl.ds(base, LANES)]
                t16 = tgt_v[pl.ds(base, LANES)]
                w16 = w_v[pl.ds(base, LANES)]

                def edge(jj, carry2):
                    j = LANES - 1 - jj
                    t_b = bcast(t16, j)
                    s_b = bcast(s16, j)
                    w_b = bcast(w16, j)
                    ct = plsc.load_gather(c_v, [t_b])
                    plsc.addupdate_scatter(c_v, [s_b], w_b * ct, mask=lane0)
                    return carry2

                lax.fori_loop(0, LANES, edge, 0)
                return carry

            lax.fori_loop(0, E // LANES, body, 0)
            pltpu.sync_copy(c_v, c_hbm)

    return scan_kernel(edge_index, w)


NB = 32


def _tanh_terms_tc(x, W, b):
    def body(x_ref, w_ref, b_ref, t_ref):
        for k in range(NB):
            t_ref[k] = jnp.tanh(
                jnp.dot(x_ref[...], w_ref[k], preferred_element_type=jnp.float32)
                + b_ref[k][None, :]
            ).astype(jnp.bfloat16)

    return pl.pallas_call(
        body,
        grid=(N // NB,),
        in_specs=[
            pl.BlockSpec((B, D_IN), lambda i: (0, 0)),
            pl.BlockSpec((NB, D_IN, D_H), lambda i: (i, 0, 0)),
            pl.BlockSpec((NB, D_H), lambda i: (i, 0)),
        ],
        out_specs=pl.BlockSpec((NB, B, D_H), lambda i: (i, 0, 0)),
        out_shape=jax.ShapeDtypeStruct((N, B, D_H), jnp.bfloat16),
        compiler_params=pltpu.CompilerParams(
            dimension_semantics=("arbitrary",),
        ),
    )(x, W, b)


def _weighted_reduce_tc(c, T):
    def body(c_ref, t_ref, out_ref):
        i = pl.program_id(0)

        @pl.when(i == 0)
        def _():
            out_ref[...] = jnp.zeros_like(out_ref)

        acc = out_ref[...]
        for k in range(NB):
            acc = acc + c_ref[i * NB + k] * t_ref[k].astype(jnp.float32)
        out_ref[...] = acc

    return pl.pallas_call(
        body,
        grid=(N // NB,),
        in_specs=[
            pl.BlockSpec(memory_space=pltpu.SMEM),
            pl.BlockSpec((NB, B, D_H), lambda i: (i, 0, 0)),
        ],
        out_specs=pl.BlockSpec((B, D_H), lambda i: (0, 0)),
        out_shape=jax.ShapeDtypeStruct((B, D_H), jnp.float32),
        compiler_params=pltpu.CompilerParams(
            dimension_semantics=("arbitrary",),
        ),
    )(c, T)


def kernel(x, W, b, edge_index, edge_weights):
    T = _tanh_terms_tc(x, W, b)
    c = _edge_scan_sc(edge_index, edge_weights)
    return _weighted_reduce_tc(c, T)
